# baseline (speedup 1.0000x reference)
import functools

import jax
import jax.numpy as jnp
from jax import lax
from jax.experimental import pallas as pl
from jax.experimental.pallas import tpu as pltpu
from jax.experimental.pallas import tpu_sc as plsc

B = 16384
D = 16
L = 16
NC = 2
NS = 16
NW = NC * NS
CHUNK = B // NW
SUB = 128
NSUB = CHUNK // SUB


def _gmf_sc(gi2, si2, gtab, stab, wf, bf):
    mesh = plsc.VectorSubcoreMesh(core_axis_name="c", subcore_axis_name="s")

    @functools.partial(
        pl.kernel,
        mesh=mesh,
        out_type=jax.ShapeDtypeStruct((B,), jnp.float32),
        compiler_params=pltpu.CompilerParams(
            needs_layout_passes=False, use_tc_tiling_on_sc=False),
        scratch_types=[
            pltpu.VMEM((NSUB, SUB), jnp.int32),
            pltpu.VMEM((NSUB, SUB), jnp.int32),
            pltpu.VMEM((CHUNK, D), jnp.float32),
            pltpu.VMEM((CHUNK, D), jnp.float32),
            pltpu.VMEM((CHUNK,), jnp.float32),
            pltpu.VMEM((D,), jnp.float32),
            pltpu.VMEM((L,), jnp.float32),
            pltpu.SemaphoreType.DMA,
        ],
    )
    def gmf(gidx_hbm, sidx_hbm, gtab_hbm, stab_hbm, w_hbm, b_hbm, out_hbm,
            gidx_v, sidx_v, g_v, s_v, out_v, w_v, b_v, sem):
        wid = lax.axis_index("s") * NC + lax.axis_index("c")
        base = wid * CHUNK
        pltpu.sync_copy(gidx_hbm.at[wid], gidx_v)
        pltpu.sync_copy(sidx_hbm.at[wid], sidx_v)
        copies = []
        for j in range(NSUB):
            copies.append(pltpu.async_copy(
                gtab_hbm.at[gidx_v.at[j]], g_v.at[pl.ds(j * SUB, SUB)], sem))
            copies.append(pltpu.async_copy(
                stab_hbm.at[sidx_v.at[j]], s_v.at[pl.ds(j * SUB, SUB)], sem))
        pltpu.sync_copy(w_hbm, w_v)
        pltpu.sync_copy(b_hbm, b_v)
        for c in copies:
            c.wait()

        bvec = b_v[...]
        wvec = w_v[...]
        lanes = lax.iota(jnp.int32, L)
        cvecs = [jnp.full((L,), d, dtype=jnp.int32) for d in range(D)]

        def scale_body(i, carry):
            g_v[i] = g_v[i] * wvec
            return carry

        lax.fori_loop(0, CHUNK, scale_body, 0)

        def body(blk, carry):
            row0 = blk * L
            ridx = lanes + row0
            acc = bvec
            for d in range(D):
                gcol = plsc.load_gather(g_v, [ridx, cvecs[d]])
                scol = plsc.load_gather(s_v, [ridx, cvecs[d]])
                acc = acc + gcol * scol
            out_v[pl.ds(row0, L)] = acc
            return carry

        lax.fori_loop(0, CHUNK // L, body, 0)
        pltpu.sync_copy(out_v, out_hbm.at[pl.ds(base, CHUNK)])

    return gmf(gi2, si2, gtab, stab, wf, bf)


def kernel(gene_indices, spot_indices, gene_table, spot_table, W, b):
    gi2 = gene_indices.astype(jnp.int32).reshape(NW, NSUB, SUB)
    si2 = spot_indices.astype(jnp.int32).reshape(NW, NSUB, SUB)
    wf = W.reshape(D).astype(jnp.float32)
    bf = jnp.broadcast_to(b.astype(jnp.float32), (L,))
    return _gmf_sc(gi2, si2, gene_table, spot_table, wf, bf)


# speedup vs baseline: 5.7058x; 5.7058x over previous
import functools

import jax
import jax.numpy as jnp
from jax import lax
from jax.experimental import pallas as pl
from jax.experimental.pallas import tpu as pltpu
from jax.experimental.pallas import tpu_sc as plsc

B = 16384
D = 16
L = 16
NC = 2
NS = 16
NW = NC * NS
CHUNK = B // NW
NGRP = CHUNK // L


def _splat(vec, lane):
    return lax.gather(
        vec, jnp.full((L, 1), lane, jnp.int32),
        lax.GatherDimensionNumbers(
            offset_dims=(), collapsed_slice_dims=(0,), start_index_map=(0,)),
        (1,), mode=lax.GatherScatterMode.PROMISE_IN_BOUNDS)


def _gmf_sc(gi2, si2, gtabT, stabT, wf, bf):
    mesh = plsc.VectorSubcoreMesh(core_axis_name="c", subcore_axis_name="s")

    @functools.partial(
        pl.kernel,
        mesh=mesh,
        out_type=jax.ShapeDtypeStruct((B,), jnp.float32),
        compiler_params=pltpu.CompilerParams(
            needs_layout_passes=False,
            use_tc_tiling_on_sc=True,
            disable_bounds_checks=True,
        ),
        scratch_types=[
            pltpu.VMEM((4, 128), jnp.int32),
            pltpu.VMEM((4, 128), jnp.int32),
            pltpu.VMEM((2, L, D, 128), jnp.float32),
            pltpu.VMEM((D, CHUNK), jnp.float32),
            pltpu.VMEM((D, CHUNK), jnp.float32),
            pltpu.VMEM((L,), jnp.float32),
            pltpu.VMEM((D,), jnp.float32),
            pltpu.VMEM((CHUNK,), jnp.float32),
            pltpu.SemaphoreType.DMA,
            pltpu.SemaphoreType.DMA,
        ],
    )
    def gmf(gidx_hbm, sidx_hbm, gtabT_hbm, stabT_hbm, w_hbm, b_hbm, out_hbm,
            gidx_v, sidx_v, buf, gT, sT, b_v, w_v, out_v, sem0, sem1):
        wid = lax.axis_index("s") * NC + lax.axis_index("c")
        base = wid * CHUNK
        pltpu.sync_copy(gidx_hbm.at[wid], gidx_v)
        pltpu.sync_copy(sidx_hbm.at[wid], sidx_v)
        pltpu.sync_copy(w_hbm, w_v)
        pltpu.sync_copy(b_hbm, b_v)
        lanes = lax.iota(jnp.int32, L)
        sems = (sem0, sem1)

        def group_vec(idx_v, g):
            c = lax.shift_right_logical(g, 3)
            k0 = lax.shift_left(jnp.bitwise_and(g, 7), 4)
            return idx_v[c, pl.ds(k0, L)]

        def issue_group(tab, idx_v, g, par):
            v = group_vec(idx_v, g)
            cps = []
            for e in range(L):
                off = lax.shift_left(lax.shift_right_logical(v[e], 7), 7)
                off = pl.multiple_of(off, 128)
                cps.append(pltpu.async_copy(
                    tab.at[:, pl.ds(off, 128)], buf.at[par, e], sems[par]))
            return cps

        def extract_group(idx_v, cT, g, par, wsp):
            rlo = jnp.bitwise_and(group_vec(idx_v, g), jnp.int32(127))
            pvec = jnp.full((L,), par, dtype=jnp.int32)
            o = lax.shift_left(g, 4)
            for d in range(D):
                dvec = jnp.full((L,), d, dtype=jnp.int32)
                val = plsc.load_gather(buf, [pvec, lanes, dvec, rlo])
                if wsp is not None:
                    val = val * wsp[d]
                cT[d, pl.ds(o, L)] = val

        def phase(tab, idx_v, cT, wsp):
            c0 = issue_group(tab, idx_v, jnp.int32(0), 0)

            def body(h, carry):
                g0 = lax.shift_left(h, 1)
                c1 = issue_group(tab, idx_v, g0 + 1, 1)
                for cp in c0:
                    cp.wait()
                extract_group(idx_v, cT, g0, 0, wsp)
                nxt = jnp.where(g0 + 2 < NGRP, g0 + 2, jnp.int32(0))
                issue_group(tab, idx_v, nxt, 0)
                for cp in c1:
                    cp.wait()
                extract_group(idx_v, cT, g0 + 1, 1, wsp)
                return carry

            lax.fori_loop(0, NGRP // 2, body, 0)
            dummy = pltpu.make_async_copy(
                tab.at[:, pl.ds(0, 128)], buf.at[0, 0], sems[0])
            for _ in range(L):
                dummy.wait()

        wvec = w_v[...]
        wsp = [_splat(wvec, d) for d in range(D)]
        phase(gtabT_hbm, gidx_v, gT, wsp)
        phase(stabT_hbm, sidx_v, sT, None)

        bvec = b_v[...]

        def mac(blk, carry):
            o = lax.shift_left(blk, 4)
            acc = bvec
            for d in range(D):
                acc = acc + gT[d, pl.ds(o, L)] * sT[d, pl.ds(o, L)]
            out_v[pl.ds(o, L)] = acc
            return carry

        lax.fori_loop(0, NGRP, mac, 0)
        pltpu.sync_copy(out_v, out_hbm.at[pl.ds(base, CHUNK)])

    return gmf(gi2, si2, gtabT, stabT, wf, bf)


def kernel(gene_indices, spot_indices, gene_table, spot_table, W, b):
    gi2 = gene_indices.astype(jnp.int32).reshape(NW, 4, 128)
    si2 = spot_indices.astype(jnp.int32).reshape(NW, 4, 128)
    wf = W.reshape(D).astype(jnp.float32)
    bf = jnp.broadcast_to(b.astype(jnp.float32), (L,))
    return _gmf_sc(gi2, si2, gene_table.T, spot_table.T, wf, bf)


# speedup vs baseline: 5.8448x; 1.0244x over previous
import functools

import jax
import jax.numpy as jnp
from jax import lax
from jax.experimental import pallas as pl
from jax.experimental.pallas import tpu as pltpu
from jax.experimental.pallas import tpu_sc as plsc

B = 16384
D = 16
L = 16
NC = 2
NS = 16
NW = NC * NS
CHUNK = B // NW
NGRP = CHUNK // L


def _splat(vec, lane):
    return lax.gather(
        vec, jnp.full((L, 1), lane, jnp.int32),
        lax.GatherDimensionNumbers(
            offset_dims=(), collapsed_slice_dims=(0,), start_index_map=(0,)),
        (1,), mode=lax.GatherScatterMode.PROMISE_IN_BOUNDS)


def _gmf_sc(gi2, si2, gtabT, stabT, wf, bf):
    mesh = plsc.VectorSubcoreMesh(core_axis_name="c", subcore_axis_name="s")

    @functools.partial(
        pl.kernel,
        mesh=mesh,
        out_type=jax.ShapeDtypeStruct((B,), jnp.float32),
        compiler_params=pltpu.CompilerParams(
            needs_layout_passes=False,
            use_tc_tiling_on_sc=True,
            disable_bounds_checks=True,
        ),
        scratch_types=[
            pltpu.VMEM((4, 128), jnp.int32),
            pltpu.VMEM((4, 128), jnp.int32),
            pltpu.VMEM((2, L, D, 128), jnp.float32),
            pltpu.VMEM((D, CHUNK), jnp.float32),
            pltpu.VMEM((D, CHUNK), jnp.float32),
            pltpu.VMEM((L,), jnp.float32),
            pltpu.VMEM((D,), jnp.float32),
            pltpu.VMEM((CHUNK,), jnp.float32),
            pltpu.SemaphoreType.DMA,
            pltpu.SemaphoreType.DMA,
        ],
    )
    def gmf(gidx_hbm, sidx_hbm, gtabT_hbm, stabT_hbm, w_hbm, b_hbm, out_hbm,
            gidx_v, sidx_v, buf, gT, sT, b_v, w_v, out_v, sem0, sem1):
        wid = lax.axis_index("s") * NC + lax.axis_index("c")
        base = wid * CHUNK
        pltpu.sync_copy(gidx_hbm.at[wid], gidx_v)
        pltpu.sync_copy(sidx_hbm.at[wid], sidx_v)
        pltpu.sync_copy(w_hbm, w_v)
        pltpu.sync_copy(b_hbm, b_v)
        lanes = lax.iota(jnp.int32, L)
        sems = (sem0, sem1)

        def group_vec(idx_v, g):
            c = lax.shift_right_logical(g, 3)
            k0 = lax.shift_left(jnp.bitwise_and(g, 7), 4)
            return idx_v[c, pl.ds(k0, L)]

        def issue_group(tab, idx_v, g, par):
            offv = lax.shift_left(
                lax.shift_right_logical(group_vec(idx_v, g), 7), 7)
            cps = []
            for e in range(L):
                off = pl.multiple_of(offv[e], 128)
                cps.append(pltpu.async_copy(
                    tab.at[:, pl.ds(off, 128)], buf.at[par, e], sems[par]))
            return cps

        def extract_group(idx_v, cT, g, par, wsp):
            rlo = jnp.bitwise_and(group_vec(idx_v, g), jnp.int32(127))
            pvec = jnp.full((L,), par, dtype=jnp.int32)
            o = lax.shift_left(g, 4)
            for d in range(D):
                dvec = jnp.full((L,), d, dtype=jnp.int32)
                val = plsc.load_gather(buf, [pvec, lanes, dvec, rlo])
                if wsp is not None:
                    val = val * wsp[d]
                cT[d, pl.ds(o, L)] = val

        wvec = w_v[...]
        wsp = [_splat(wvec, d) for d in range(D)]

        c0 = issue_group(gtabT_hbm, gidx_v, jnp.int32(0), 0)

        def body(g, carry):
            c1 = issue_group(stabT_hbm, sidx_v, g, 1)
            for cp in c0:
                cp.wait()
            extract_group(gidx_v, gT, g, 0, wsp)
            nxtg = jnp.where(g + 1 < NGRP, g + 1, jnp.int32(0))
            issue_group(gtabT_hbm, gidx_v, nxtg, 0)
            for cp in c1:
                cp.wait()
            extract_group(sidx_v, sT, g, 1, None)
            return carry

        lax.fori_loop(0, NGRP, body, 0)
        dummy = pltpu.make_async_copy(
            gtabT_hbm.at[:, pl.ds(0, 128)], buf.at[0, 0], sems[0])
        for _ in range(L):
            dummy.wait()

        bvec = b_v[...]

        def mac(blk, carry):
            o = lax.shift_left(blk, 4)
            acc = bvec
            for d in range(D):
                acc = acc + gT[d, pl.ds(o, L)] * sT[d, pl.ds(o, L)]
            out_v[pl.ds(o, L)] = acc
            return carry

        lax.fori_loop(0, NGRP, mac, 0)
        pltpu.sync_copy(out_v, out_hbm.at[pl.ds(base, CHUNK)])

    return gmf(gi2, si2, gtabT, stabT, wf, bf)


def kernel(gene_indices, spot_indices, gene_table, spot_table, W, b):
    gi2 = gene_indices.astype(jnp.int32).reshape(NW, 4, 128)
    si2 = spot_indices.astype(jnp.int32).reshape(NW, 4, 128)
    wf = W.reshape(D).astype(jnp.float32)
    bf = jnp.broadcast_to(b.astype(jnp.float32), (L,))
    return _gmf_sc(gi2, si2, gene_table.T, spot_table.T, wf, bf)


# speedup vs baseline: 6.1510x; 1.0524x over previous
import functools

import jax
import jax.numpy as jnp
from jax import lax
from jax.experimental import pallas as pl
from jax.experimental.pallas import tpu as pltpu
from jax.experimental.pallas import tpu_sc as plsc

B = 16384
D = 16
L = 16
NC = 2
NS = 16
NW = NC * NS
CHUNK = B // NW
NGRP = CHUNK // L


def _splat(vec, lane):
    return lax.gather(
        vec, jnp.full((L, 1), lane, jnp.int32),
        lax.GatherDimensionNumbers(
            offset_dims=(), collapsed_slice_dims=(0,), start_index_map=(0,)),
        (1,), mode=lax.GatherScatterMode.PROMISE_IN_BOUNDS)


def _gmf_sc(gi2, si2, gtabT, stabT, wf, bf):
    mesh = plsc.VectorSubcoreMesh(core_axis_name="c", subcore_axis_name="s")

    @functools.partial(
        pl.kernel,
        mesh=mesh,
        out_type=jax.ShapeDtypeStruct((B,), jnp.float32),
        compiler_params=pltpu.CompilerParams(
            needs_layout_passes=False,
            use_tc_tiling_on_sc=True,
            disable_bounds_checks=True,
        ),
        scratch_types=[
            pltpu.VMEM((4, 128), jnp.int32),
            pltpu.VMEM((4, 128), jnp.int32),
            pltpu.VMEM((3, L, D, 128), jnp.float32),
            pltpu.VMEM((D, CHUNK), jnp.float32),
            pltpu.VMEM((D, CHUNK), jnp.float32),
            pltpu.VMEM((L,), jnp.float32),
            pltpu.VMEM((D,), jnp.float32),
            pltpu.VMEM((CHUNK,), jnp.float32),
            pltpu.SemaphoreType.DMA,
            pltpu.SemaphoreType.DMA,
            pltpu.SemaphoreType.DMA,
        ],
    )
    def gmf(gidx_hbm, sidx_hbm, gtabT_hbm, stabT_hbm, w_hbm, b_hbm, out_hbm,
            gidx_v, sidx_v, buf, gT, sT, b_v, w_v, out_v, sem0, sem1,
            sem2):
        wid = lax.axis_index("s") * NC + lax.axis_index("c")
        base = wid * CHUNK
        pltpu.sync_copy(gidx_hbm.at[wid], gidx_v)
        pltpu.sync_copy(sidx_hbm.at[wid], sidx_v)
        pltpu.sync_copy(w_hbm, w_v)
        pltpu.sync_copy(b_hbm, b_v)
        lanes = lax.iota(jnp.int32, L)
        sems = (sem0, sem1, sem2)

        def group_vec(idx_v, g):
            c = lax.shift_right_logical(g, 3)
            k0 = lax.shift_left(jnp.bitwise_and(g, 7), 4)
            return idx_v[c, pl.ds(k0, L)]

        def issue_group(tab, idx_v, g, par):
            offv = lax.shift_left(
                lax.shift_right_logical(group_vec(idx_v, g), 7), 7)
            cps = []
            for e in range(L):
                off = pl.multiple_of(offv[e], 128)
                cps.append(pltpu.async_copy(
                    tab.at[:, pl.ds(off, 128)], buf.at[par, e], sems[par]))
            return cps

        def extract_group(idx_v, cT, g, par, wsp):
            rlo = jnp.bitwise_and(group_vec(idx_v, g), jnp.int32(127))
            pvec = jnp.full((L,), par, dtype=jnp.int32)
            o = lax.shift_left(g, 4)
            for d in range(D):
                dvec = jnp.full((L,), d, dtype=jnp.int32)
                val = plsc.load_gather(buf, [pvec, lanes, dvec, rlo])
                if wsp is not None:
                    val = val * wsp[d]
                cT[d, pl.ds(o, L)] = val

        wvec = w_v[...]
        wsp = [_splat(wvec, d) for d in range(D)]

        issue_group(gtabT_hbm, gidx_v, jnp.int32(0), 0)
        issue_group(stabT_hbm, sidx_v, jnp.int32(0), 1)
        wait16 = [
            pltpu.make_async_copy(
                gtabT_hbm.at[:, pl.ds(0, 128)], buf.at[sl, 0], sems[sl])
            for sl in range(3)
        ]

        def drain(sl):
            for _ in range(L):
                wait16[sl].wait()

        def body(h, carry):
            g0 = h * 3
            nx = [jnp.where(g + 1 < NGRP, g + 1, jnp.int32(0))
                  for g in (g0, g0 + 1, g0 + 2)]
            issue_group(gtabT_hbm, gidx_v, g0 + 1, 2)
            drain(0)
            extract_group(gidx_v, gT, g0, 0, wsp)
            issue_group(stabT_hbm, sidx_v, g0 + 1, 0)
            drain(1)
            extract_group(sidx_v, sT, g0, 1, None)
            issue_group(gtabT_hbm, gidx_v, nx[1], 1)
            drain(2)
            extract_group(gidx_v, gT, g0 + 1, 2, wsp)
            issue_group(stabT_hbm, sidx_v, nx[1], 2)
            drain(0)
            extract_group(sidx_v, sT, g0 + 1, 0, None)
            issue_group(gtabT_hbm, gidx_v, nx[2], 0)
            drain(1)
            extract_group(gidx_v, gT, g0 + 2, 1, wsp)
            issue_group(stabT_hbm, sidx_v, nx[2], 1)
            drain(2)
            extract_group(sidx_v, sT, g0 + 2, 2, None)
            return carry

        lax.fori_loop(0, NGRP // 3, body, 0)
        drain(0)
        extract_group(gidx_v, gT, jnp.int32(30), 0, wsp)
        issue_group(gtabT_hbm, gidx_v, jnp.int32(31), 2)
        drain(1)
        extract_group(sidx_v, sT, jnp.int32(30), 1, None)
        issue_group(stabT_hbm, sidx_v, jnp.int32(31), 0)
        drain(2)
        extract_group(gidx_v, gT, jnp.int32(31), 2, wsp)
        drain(0)
        extract_group(sidx_v, sT, jnp.int32(31), 0, None)

        bvec = b_v[...]

        def mac(blk, carry):
            o = lax.shift_left(blk, 4)
            acc = bvec
            for d in range(D):
                acc = acc + gT[d, pl.ds(o, L)] * sT[d, pl.ds(o, L)]
            out_v[pl.ds(o, L)] = acc
            return carry

        lax.fori_loop(0, NGRP, mac, 0)
        pltpu.sync_copy(out_v, out_hbm.at[pl.ds(base, CHUNK)])

    return gmf(gi2, si2, gtabT, stabT, wf, bf)


def kernel(gene_indices, spot_indices, gene_table, spot_table, W, b):
    gi2 = gene_indices.astype(jnp.int32).reshape(NW, 4, 128)
    si2 = spot_indices.astype(jnp.int32).reshape(NW, 4, 128)
    wf = W.reshape(D).astype(jnp.float32)
    bf = jnp.broadcast_to(b.astype(jnp.float32), (L,))
    return _gmf_sc(gi2, si2, gene_table.T, spot_table.T, wf, bf)


# speedup vs baseline: 6.1810x; 1.0049x over previous
import functools

import jax
import jax.numpy as jnp
from jax import lax
from jax.experimental import pallas as pl
from jax.experimental.pallas import tpu as pltpu
from jax.experimental.pallas import tpu_sc as plsc

B = 16384
D = 16
L = 16
NC = 2
NS = 16
NW = NC * NS
CHUNK = B // NW
NGRP = CHUNK // L


def _splat(vec, lane):
    return lax.gather(
        vec, jnp.full((L, 1), lane, jnp.int32),
        lax.GatherDimensionNumbers(
            offset_dims=(), collapsed_slice_dims=(0,), start_index_map=(0,)),
        (1,), mode=lax.GatherScatterMode.PROMISE_IN_BOUNDS)


def _gmf_sc(gi2, si2, gtabT, stabT, wf, bf):
    mesh = plsc.VectorSubcoreMesh(core_axis_name="c", subcore_axis_name="s")

    @functools.partial(
        pl.kernel,
        mesh=mesh,
        out_type=jax.ShapeDtypeStruct((B,), jnp.float32),
        compiler_params=pltpu.CompilerParams(
            needs_layout_passes=False,
            use_tc_tiling_on_sc=True,
            disable_bounds_checks=True,
        ),
        scratch_types=[
            pltpu.VMEM((4, 128), jnp.int32),
            pltpu.VMEM((4, 128), jnp.int32),
            pltpu.VMEM((3 * L * D, 128), jnp.float32),
            pltpu.VMEM((D, CHUNK), jnp.float32),
            pltpu.VMEM((D, CHUNK), jnp.float32),
            pltpu.VMEM((L,), jnp.float32),
            pltpu.VMEM((D,), jnp.float32),
            pltpu.VMEM((CHUNK,), jnp.float32),
            pltpu.SemaphoreType.DMA,
            pltpu.SemaphoreType.DMA,
            pltpu.SemaphoreType.DMA,
        ],
    )
    def gmf(gidx_hbm, sidx_hbm, gtabT_hbm, stabT_hbm, w_hbm, b_hbm, out_hbm,
            gidx_v, sidx_v, buf, gT, sT, b_v, w_v, out_v, sem0, sem1,
            sem2):
        wid = lax.axis_index("s") * NC + lax.axis_index("c")
        base = wid * CHUNK
        pltpu.sync_copy(gidx_hbm.at[wid], gidx_v)
        pltpu.sync_copy(sidx_hbm.at[wid], sidx_v)
        pltpu.sync_copy(w_hbm, w_v)
        pltpu.sync_copy(b_hbm, b_v)
        lanes = lax.iota(jnp.int32, L)
        sems = (sem0, sem1, sem2)

        def group_vec(idx_v, g):
            c = lax.shift_right_logical(g, 3)
            k0 = lax.shift_left(jnp.bitwise_and(g, 7), 4)
            return idx_v[c, pl.ds(k0, L)]

        def issue_group(tab, idx_v, g, par):
            offv = lax.shift_left(
                lax.shift_right_logical(group_vec(idx_v, g), 7), 7)
            cps = []
            for e in range(L):
                off = pl.multiple_of(offv[e], 128)
                cps.append(pltpu.async_copy(
                    tab.at[:, pl.ds(off, 128)],
                    buf.at[pl.ds((par * L + e) * D, D)], sems[par]))
            return cps

        rowbase = lax.shift_left(lanes, 4)

        def extract_group(idx_v, cT, g, par, wsp):
            rlo = jnp.bitwise_and(group_vec(idx_v, g), jnp.int32(127))
            rb = rowbase + par * L * D
            o = lax.shift_left(g, 4)
            for d in range(D):
                val = plsc.load_gather(buf, [rb + d, rlo])
                if wsp is not None:
                    val = val * wsp[d]
                cT[d, pl.ds(o, L)] = val

        wvec = w_v[...]
        wsp = [_splat(wvec, d) for d in range(D)]

        issue_group(gtabT_hbm, gidx_v, jnp.int32(0), 0)
        issue_group(stabT_hbm, sidx_v, jnp.int32(0), 1)
        wait16 = [
            pltpu.make_async_copy(
                gtabT_hbm.at[:, pl.ds(0, 128)],
                buf.at[pl.ds(sl * L * D, D)], sems[sl])
            for sl in range(3)
        ]

        def drain(sl):
            for _ in range(L):
                wait16[sl].wait()

        def body(h, carry):
            g0 = h * 3
            nx = [jnp.where(g + 1 < NGRP, g + 1, jnp.int32(0))
                  for g in (g0, g0 + 1, g0 + 2)]
            issue_group(gtabT_hbm, gidx_v, g0 + 1, 2)
            drain(0)
            extract_group(gidx_v, gT, g0, 0, wsp)
            issue_group(stabT_hbm, sidx_v, g0 + 1, 0)
            drain(1)
            extract_group(sidx_v, sT, g0, 1, None)
            issue_group(gtabT_hbm, gidx_v, nx[1], 1)
            drain(2)
            extract_group(gidx_v, gT, g0 + 1, 2, wsp)
            issue_group(stabT_hbm, sidx_v, nx[1], 2)
            drain(0)
            extract_group(sidx_v, sT, g0 + 1, 0, None)
            issue_group(gtabT_hbm, gidx_v, nx[2], 0)
            drain(1)
            extract_group(gidx_v, gT, g0 + 2, 1, wsp)
            issue_group(stabT_hbm, sidx_v, nx[2], 1)
            drain(2)
            extract_group(sidx_v, sT, g0 + 2, 2, None)
            return carry

        lax.fori_loop(0, NGRP // 3, body, 0)
        drain(0)
        extract_group(gidx_v, gT, jnp.int32(30), 0, wsp)
        issue_group(gtabT_hbm, gidx_v, jnp.int32(31), 2)
        drain(1)
        extract_group(sidx_v, sT, jnp.int32(30), 1, None)
        issue_group(stabT_hbm, sidx_v, jnp.int32(31), 0)
        drain(2)
        extract_group(gidx_v, gT, jnp.int32(31), 2, wsp)
        drain(0)
        extract_group(sidx_v, sT, jnp.int32(31), 0, None)

        bvec = b_v[...]

        def mac(blk, carry):
            o = lax.shift_left(blk, 4)
            acc = bvec
            for d in range(D):
                acc = acc + gT[d, pl.ds(o, L)] * sT[d, pl.ds(o, L)]
            out_v[pl.ds(o, L)] = acc
            return carry

        lax.fori_loop(0, NGRP, mac, 0)
        pltpu.sync_copy(out_v, out_hbm.at[pl.ds(base, CHUNK)])

    return gmf(gi2, si2, gtabT, stabT, wf, bf)


def kernel(gene_indices, spot_indices, gene_table, spot_table, W, b):
    gi2 = gene_indices.astype(jnp.int32).reshape(NW, 4, 128)
    si2 = spot_indices.astype(jnp.int32).reshape(NW, 4, 128)
    wf = W.reshape(D).astype(jnp.float32)
    bf = jnp.broadcast_to(b.astype(jnp.float32), (L,))
    return _gmf_sc(gi2, si2, gene_table.T, spot_table.T, wf, bf)
